# Initial kernel scaffold; baseline (speedup 1.0000x reference)
import jax
import jax.numpy as jnp
from jax.experimental import pallas as pl


def kernel(x, up_index, up_attr, down_index, down_attr):
    raise NotImplementedError("write your pallas kernel here")


# speedup vs baseline: 4.5537x; 4.5537x over previous
import functools

import jax
import jax.numpy as jnp
from jax import lax
from jax.experimental import pallas as pl
from jax.experimental.pallas import tpu as pltpu
from jax.experimental.pallas import tpu_sc as plsc

N = 10000
E = 320000
D = 128

NC = 2
NS = 16
NW = NC * NS
EPW = E // NW
CH = 80
NCH = EPW // CH
IB = 25
NB = NCH // IB
N_PAD = 10112
RPT = N_PAD // NS


def _sc_body(x_hbm, us_hbm, ud_hbm, ua_hbm, ds_hbm, dd_hbm, da_hbm, z_hbm,
             out0, out1,
             acc, idx_src, idx_dst, xbuf, abuf, sem_g, sem_a):
    c = lax.axis_index("c")
    s = lax.axis_index("s")
    w = s * NC + c
    t = s

    pltpu.sync_copy(z_hbm, acc.at[pl.ds(t * RPT, RPT)])
    plsc.subcore_barrier()

    def make_block_body(src_hbm, dst_hbm, attr_hbm):
        def chunk_body(b, j, carry):
            h1 = pltpu.async_copy(x_hbm.at[idx_src.at[j]], xbuf, sem_g)
            h2 = pltpu.async_copy(
                attr_hbm.at[w, pl.ds((b * IB + j) * CH, CH)], abuf, sem_a)
            h1.wait()
            pltpu.sync_copy(xbuf, acc.at[idx_dst.at[j]], add=True)
            h2.wait()
            pltpu.sync_copy(abuf, acc.at[idx_dst.at[j]], add=True)
            return carry

        def block_body(b, carry):
            pltpu.sync_copy(src_hbm.at[w, b], idx_src)
            pltpu.sync_copy(dst_hbm.at[w, b], idx_dst)
            return lax.fori_loop(0, IB, lambda j, cr: chunk_body(b, j, cr),
                                 carry)
        return block_body

    lax.fori_loop(0, NB, make_block_body(us_hbm, ud_hbm, ua_hbm), 0)
    lax.fori_loop(0, NB, make_block_body(ds_hbm, dd_hbm, da_hbm), 0)

    plsc.subcore_barrier()

    @pl.when(c == 0)
    def _():
        pltpu.sync_copy(acc.at[pl.ds(t * RPT, RPT)],
                        out0.at[pl.ds(t * RPT, RPT)])

    @pl.when(c == 1)
    def _():
        pltpu.sync_copy(acc.at[pl.ds(t * RPT, RPT)],
                        out1.at[pl.ds(t * RPT, RPT)])


def _combine_body(x_ref, a_ref, b_ref, o_ref):
    o_ref[...] = x_ref[...] + a_ref[...] + b_ref[...]


def kernel(x, up_index, up_attr, down_index, down_attr):
    us = up_index[0].astype(jnp.int32).reshape(NW, NB, IB, CH)
    ud = up_index[1].astype(jnp.int32).reshape(NW, NB, IB, CH)
    ds_ = down_index[0].astype(jnp.int32).reshape(NW, NB, IB, CH)
    dd = down_index[1].astype(jnp.int32).reshape(NW, NB, IB, CH)
    ua = up_attr.reshape(NW, EPW, D)
    da = down_attr.reshape(NW, EPW, D)
    zeros = jnp.zeros((RPT, D), jnp.float32)

    mesh = plsc.VectorSubcoreMesh(core_axis_name="c", subcore_axis_name="s")
    scatter = pl.kernel(
        _sc_body,
        mesh=mesh,
        out_type=[jax.ShapeDtypeStruct((N_PAD, D), jnp.float32),
                  jax.ShapeDtypeStruct((N_PAD, D), jnp.float32)],
        scratch_types=[
            pltpu.VMEM_SHARED((N_PAD, D), jnp.float32),
            pltpu.VMEM((IB, CH), jnp.int32),
            pltpu.VMEM((IB, CH), jnp.int32),
            pltpu.VMEM((CH, D), jnp.float32),
            pltpu.VMEM((CH, D), jnp.float32),
            pltpu.SemaphoreType.DMA,
            pltpu.SemaphoreType.DMA,
        ],
    )
    a0, a1 = scatter(x, us, ud, ua, ds_, dd, da, zeros)

    blk = 1000
    out = pl.pallas_call(
        _combine_body,
        grid=(N // blk,),
        in_specs=[pl.BlockSpec((blk, D), lambda i: (i, 0))] * 3,
        out_specs=pl.BlockSpec((blk, D), lambda i: (i, 0)),
        out_shape=jax.ShapeDtypeStruct((N, D), jnp.float32),
    )(x, a0, a1)
    return out


# speedup vs baseline: 6.1047x; 1.3406x over previous
import functools

import jax
import jax.numpy as jnp
from jax import lax
from jax.experimental import pallas as pl
from jax.experimental.pallas import tpu as pltpu
from jax.experimental.pallas import tpu_sc as plsc

N = 10000
E = 320000
D = 128

NC = 2
NS = 16
NW = NC * NS
EPW = E // NW
CH = 40
NCH = EPW // CH
IB = 50
NB = NCH // IB
N_PAD = 10112
RPT = N_PAD // NS


def _sc_body(x_hbm, us_hbm, ud_hbm, ua_hbm, ds_hbm, dd_hbm, da_hbm, z_hbm,
             out0, out1,
             acc, idx_src, idx_dst, xb0, ab0, xb1, ab1, sem_g, sem_a):
    c = lax.axis_index("c")
    s = lax.axis_index("s")
    w = s * NC + c
    t = s

    pltpu.sync_copy(z_hbm, acc.at[pl.ds(t * RPT, RPT)])
    plsc.subcore_barrier()

    def make_block_body(src_hbm, dst_hbm, attr_hbm):
        def start(b, j, xb, ab):
            pltpu.async_copy(x_hbm.at[idx_src.at[j]], xb, sem_g)
            pltpu.async_copy(
                attr_hbm.at[w, pl.ds((b * IB + j) * CH, CH)], ab, sem_a)

        def drain_scatter(b, j, xb, ab):
            pltpu.make_async_copy(x_hbm.at[idx_src.at[j]], xb, sem_g).wait()
            pltpu.sync_copy(xb, acc.at[idx_dst.at[j]], add=True)
            pltpu.make_async_copy(
                attr_hbm.at[w, pl.ds((b * IB + j) * CH, CH)], ab,
                sem_a).wait()
            pltpu.sync_copy(ab, acc.at[idx_dst.at[j]], add=True)

        def pair_body(b, i, carry):
            start(b, 2 * i + 1, xb1, ab1)
            drain_scatter(b, 2 * i, xb0, ab0)

            @pl.when(i < IB // 2 - 1)
            def _():
                start(b, 2 * i + 2, xb0, ab0)
            drain_scatter(b, 2 * i + 1, xb1, ab1)
            return carry

        def block_body(b, carry):
            pltpu.sync_copy(src_hbm.at[w, b], idx_src)
            pltpu.sync_copy(dst_hbm.at[w, b], idx_dst)
            start(b, 0, xb0, ab0)
            return lax.fori_loop(0, IB // 2,
                                 lambda i, cr: pair_body(b, i, cr), carry)
        return block_body

    lax.fori_loop(0, NB, make_block_body(us_hbm, ud_hbm, ua_hbm), 0)
    lax.fori_loop(0, NB, make_block_body(ds_hbm, dd_hbm, da_hbm), 0)

    plsc.subcore_barrier()

    @pl.when(c == 0)
    def _():
        pltpu.sync_copy(acc.at[pl.ds(t * RPT, RPT)],
                        out0.at[pl.ds(t * RPT, RPT)])

    @pl.when(c == 1)
    def _():
        pltpu.sync_copy(acc.at[pl.ds(t * RPT, RPT)],
                        out1.at[pl.ds(t * RPT, RPT)])


def _combine_body(x_ref, a_ref, b_ref, o_ref):
    o_ref[...] = x_ref[...] + a_ref[...] + b_ref[...]


def kernel(x, up_index, up_attr, down_index, down_attr):
    us = up_index[0].astype(jnp.int32).reshape(NW, NB, IB, CH)
    ud = up_index[1].astype(jnp.int32).reshape(NW, NB, IB, CH)
    ds_ = down_index[0].astype(jnp.int32).reshape(NW, NB, IB, CH)
    dd = down_index[1].astype(jnp.int32).reshape(NW, NB, IB, CH)
    ua = up_attr.reshape(NW, EPW, D)
    da = down_attr.reshape(NW, EPW, D)
    zeros = jnp.zeros((RPT, D), jnp.float32)

    mesh = plsc.VectorSubcoreMesh(core_axis_name="c", subcore_axis_name="s")
    scatter = pl.kernel(
        _sc_body,
        mesh=mesh,
        out_type=[jax.ShapeDtypeStruct((N_PAD, D), jnp.float32),
                  jax.ShapeDtypeStruct((N_PAD, D), jnp.float32)],
        scratch_types=[
            pltpu.VMEM_SHARED((N_PAD, D), jnp.float32),
            pltpu.VMEM((IB, CH), jnp.int32),
            pltpu.VMEM((IB, CH), jnp.int32),
            pltpu.VMEM((CH, D), jnp.float32),
            pltpu.VMEM((CH, D), jnp.float32),
            pltpu.VMEM((CH, D), jnp.float32),
            pltpu.VMEM((CH, D), jnp.float32),
            pltpu.SemaphoreType.DMA,
            pltpu.SemaphoreType.DMA,
        ],
    )
    a0, a1 = scatter(x, us, ud, ua, ds_, dd, da, zeros)

    blk = 1000
    out = pl.pallas_call(
        _combine_body,
        grid=(N // blk,),
        in_specs=[pl.BlockSpec((blk, D), lambda i: (i, 0))] * 3,
        out_specs=pl.BlockSpec((blk, D), lambda i: (i, 0)),
        out_shape=jax.ShapeDtypeStruct((N, D), jnp.float32),
    )(x, a0, a1)
    return out


# speedup vs baseline: 6.1298x; 1.0041x over previous
import functools

import jax
import jax.numpy as jnp
from jax import lax
from jax.experimental import pallas as pl
from jax.experimental.pallas import tpu as pltpu
from jax.experimental.pallas import tpu_sc as plsc

N = 10000
E = 320000
D = 128

NC = 2
NS = 16
NW = NC * NS
NWH = NW // 2
EPW = E // NWH
CH = 80
NCH = EPW // CH
IB = 10
NB = NCH // IB
N_PAD = 10112
RPT = N_PAD // NS


def _sc_body(x_hbm, us_hbm, ud_hbm, ua_hbm, ds_hbm, dd_hbm, da_hbm, z_hbm,
             out0, out1,
             acc, idx_src, idx_dst, xb0, ab0, xb1, ab1, sem_g, sem_a):
    c = lax.axis_index("c")
    s = lax.axis_index("s")
    w = s * NC + c
    t = s

    pltpu.sync_copy(z_hbm, acc.at[pl.ds(t * RPT, RPT)])
    plsc.subcore_barrier()

    def run_pipeline(src_hbm, dst_hbm, attr_hbm, wl):
        def start(b, j, xb, ab):
            pltpu.async_copy(x_hbm.at[idx_src.at[j]], xb, sem_g)
            pltpu.async_copy(
                attr_hbm.at[wl, pl.ds((b * IB + j) * CH, CH)], ab, sem_a)

        def drain_scatter(b, j, xb, ab):
            pltpu.make_async_copy(x_hbm.at[idx_src.at[j]], xb, sem_g).wait()
            pltpu.sync_copy(xb, acc.at[idx_dst.at[j]], add=True)
            pltpu.make_async_copy(
                attr_hbm.at[wl, pl.ds((b * IB + j) * CH, CH)], ab,
                sem_a).wait()
            pltpu.sync_copy(ab, acc.at[idx_dst.at[j]], add=True)

        def pair_body(b, i, carry):
            start(b, 2 * i + 1, xb1, ab1)
            drain_scatter(b, 2 * i, xb0, ab0)

            @pl.when(i < IB // 2 - 1)
            def _():
                start(b, 2 * i + 2, xb0, ab0)
            drain_scatter(b, 2 * i + 1, xb1, ab1)
            return carry

        def block_body(b, carry):
            pltpu.sync_copy(src_hbm.at[wl, b], idx_src)
            pltpu.sync_copy(dst_hbm.at[wl, b], idx_dst)
            start(b, 0, xb0, ab0)
            return lax.fori_loop(0, IB // 2,
                                 lambda i, cr: pair_body(b, i, cr), carry)

        lax.fori_loop(0, NB, block_body, 0)

    @pl.when(w < NWH)
    def _():
        run_pipeline(us_hbm, ud_hbm, ua_hbm, w)

    @pl.when(w >= NWH)
    def _():
        run_pipeline(ds_hbm, dd_hbm, da_hbm, w - NWH)

    plsc.subcore_barrier()

    @pl.when(c == 0)
    def _():
        pltpu.sync_copy(acc.at[pl.ds(t * RPT, RPT)],
                        out0.at[pl.ds(t * RPT, RPT)])

    @pl.when(c == 1)
    def _():
        pltpu.sync_copy(acc.at[pl.ds(t * RPT, RPT)],
                        out1.at[pl.ds(t * RPT, RPT)])


def _combine_body(x_ref, a_ref, b_ref, o_ref):
    o_ref[...] = x_ref[...] + a_ref[...] + b_ref[...]


def kernel(x, up_index, up_attr, down_index, down_attr):
    us = up_index[0].astype(jnp.int32).reshape(NWH, NB, IB, CH)
    ud = up_index[1].astype(jnp.int32).reshape(NWH, NB, IB, CH)
    ds_ = down_index[0].astype(jnp.int32).reshape(NWH, NB, IB, CH)
    dd = down_index[1].astype(jnp.int32).reshape(NWH, NB, IB, CH)
    ua = up_attr.reshape(NWH, EPW, D)
    da = down_attr.reshape(NWH, EPW, D)
    zeros = jnp.zeros((RPT, D), jnp.float32)

    mesh = plsc.VectorSubcoreMesh(core_axis_name="c", subcore_axis_name="s")
    scatter = pl.kernel(
        _sc_body,
        mesh=mesh,
        out_type=[jax.ShapeDtypeStruct((N_PAD, D), jnp.float32),
                  jax.ShapeDtypeStruct((N_PAD, D), jnp.float32)],
        scratch_types=[
            pltpu.VMEM_SHARED((N_PAD, D), jnp.float32),
            pltpu.VMEM((IB, CH), jnp.int32),
            pltpu.VMEM((IB, CH), jnp.int32),
            pltpu.VMEM((CH, D), jnp.float32),
            pltpu.VMEM((CH, D), jnp.float32),
            pltpu.VMEM((CH, D), jnp.float32),
            pltpu.VMEM((CH, D), jnp.float32),
            pltpu.SemaphoreType.DMA,
            pltpu.SemaphoreType.DMA,
        ],
    )
    a0, a1 = scatter(x, us, ud, ua, ds_, dd, da, zeros)

    blk = 1000
    out = pl.pallas_call(
        _combine_body,
        grid=(N // blk,),
        in_specs=[pl.BlockSpec((blk, D), lambda i: (i, 0))] * 3,
        out_specs=pl.BlockSpec((blk, D), lambda i: (i, 0)),
        out_shape=jax.ShapeDtypeStruct((N, D), jnp.float32),
    )(x, a0, a1)
    return out
